# baseline (speedup 1.0000x reference)
import functools
import jax
import jax.numpy as jnp
from jax import lax
from jax.experimental import pallas as pl
from jax.experimental.pallas import tpu as pltpu
from jax.experimental.pallas import tpu_sc as plsc

_NC, _NS = 2, 16
_NW = _NC * _NS


def _body(emb_ref, out_ref):
    b, rows, d = out_ref.shape
    out_ref[...] = jnp.broadcast_to(emb_ref[...][None], (b, rows, d))


def kernel(x, embedding):
    B, S = x.shape
    D = embedding.shape[1]
    emb = embedding[:S]
    ROWS = 1024
    tc_part = pl.pallas_call(
        _body,
        grid=(S // ROWS,),
        in_specs=[pl.BlockSpec((ROWS, D), lambda i: (i, 0))],
        out_specs=pl.BlockSpec((B - 1, ROWS, D), lambda i: (0, i, 0)),
        out_shape=jax.ShapeDtypeStruct((B - 1, S, D), jnp.float32),
    )(emb)

    rows_w = S // _NW
    CH = 32
    n_ch = rows_w // CH
    mesh = plsc.VectorSubcoreMesh(core_axis_name="c", subcore_axis_name="s")

    @functools.partial(
        pl.kernel,
        out_type=jax.ShapeDtypeStruct((1, S, D), jnp.float32),
        mesh=mesh,
        scratch_types=[
            pltpu.VMEM((CH, D), jnp.float32),
            pltpu.VMEM((CH, D), jnp.float32),
            pltpu.SemaphoreType.DMA,
            pltpu.SemaphoreType.DMA,
            pltpu.SemaphoreType.DMA,
            pltpu.SemaphoreType.DMA,
        ],
    )
    def sc_copy(emb_hbm, out_hbm, buf0, buf1, rs0, rs1, ws0, ws1):
        wid = lax.axis_index("s") * _NC + lax.axis_index("c")
        base = wid * rows_w
        bufs, rsems, wsems = (buf0, buf1), (rs0, rs1), (ws0, ws1)

        def read(i):
            r = base + i * CH
            return pltpu.make_async_copy(
                emb_hbm.at[pl.ds(r, CH)], bufs[i % 2], rsems[i % 2])

        def write(i):
            r = base + i * CH
            return pltpu.make_async_copy(
                bufs[i % 2], out_hbm.at[0, pl.ds(r, CH)], wsems[i % 2])

        read(0).start()
        read(1).start()
        for i in range(n_ch):
            read(i).wait()
            write(i).start()
            if i + 2 < n_ch:
                write(i).wait()
                read(i + 2).start()
        write(n_ch - 2).wait()
        write(n_ch - 1).wait()

    sc_part = sc_copy(emb)
    return jnp.concatenate([tc_part, sc_part], axis=0)


# speedup vs baseline: 2.1349x; 2.1349x over previous
import functools

import jax
import jax.numpy as jnp
from jax import lax
from jax.experimental import pallas as pl
from jax.experimental.pallas import tpu as pltpu
from jax.experimental.pallas import tpu_sc as plsc

_NC, _NS = 2, 16
_NW = _NC * _NS


def kernel(x, embedding):
    B, S = x.shape
    D = embedding.shape[1]
    rows_w = S // _NW
    CH = 32
    n_pairs = rows_w // (2 * CH)

    mesh = plsc.VectorSubcoreMesh(core_axis_name="c", subcore_axis_name="s")

    @functools.partial(
        pl.kernel,
        out_type=jax.ShapeDtypeStruct((B, S, D), jnp.float32),
        mesh=mesh,
        scratch_types=(
            [pltpu.VMEM((CH, D), jnp.float32)] * 2
            + [pltpu.VMEM_SHARED((_NS, CH, D), jnp.float32)] * 2
            + [pltpu.SemaphoreType.DMA] * 8
        ),
    )
    def sc_copy(emb_hbm, out_hbm, tb0, tb1, sh0, sh1, *sems):
        rsT = sems[0:2]
        wsT = sems[2:4]
        rsS = sems[4:6]
        wsS = sems[6:8]
        cid = lax.axis_index("c")
        sid = lax.axis_index("s")
        wid = sid * _NC + cid
        base = wid * rows_w
        tbufs = (tb0, tb1)
        sbufs = (sh0.at[sid], sh1.at[sid])

        def readT(k):
            r = base + (2 * k) * CH
            return pltpu.make_async_copy(
                emb_hbm.at[pl.ds(r, CH)], tbufs[k % 2], rsT[k % 2])

        def writesT(k):
            r = base + (2 * k) * CH
            return [pltpu.make_async_copy(
                tbufs[k % 2], out_hbm.at[b, pl.ds(r, CH)], wsT[k % 2])
                for b in range(B)]

        def readS(k):
            r = base + (2 * k + 1) * CH
            return pltpu.make_async_copy(
                emb_hbm.at[pl.ds(r, CH)], sbufs[k % 2], rsS[k % 2])

        def writesS(k):
            r = base + (2 * k + 1) * CH
            return [pltpu.make_async_copy(
                sbufs[k % 2], out_hbm.at[b, pl.ds(r, CH)], wsS[k % 2])
                for b in range(B)]

        readT(0).start()
        readS(0).start()
        readT(1).start()
        readS(1).start()
        for k in range(n_pairs):
            readT(k).wait()
            for w in writesT(k):
                w.start()
            readS(k).wait()
            for w in writesS(k):
                w.start()
            if k + 2 < n_pairs:
                for w in writesT(k):
                    w.wait()
                readT(k + 2).start()
                for w in writesS(k):
                    w.wait()
                readS(k + 2).start()
        for k in (n_pairs - 2, n_pairs - 1):
            for w in writesT(k):
                w.wait()
            for w in writesS(k):
                w.wait()

    return sc_copy(embedding[:S])


# speedup vs baseline: 2.1717x; 1.0172x over previous
import functools

import jax
import jax.numpy as jnp
from jax import lax
from jax.experimental import pallas as pl
from jax.experimental.pallas import tpu as pltpu
from jax.experimental.pallas import tpu_sc as plsc

_NC, _NS = 2, 16
_NW = _NC * _NS
_NBUF = 3
_LAG = 2


def kernel(x, embedding):
    B, S = x.shape
    D = embedding.shape[1]
    rows_w = S // _NW
    CH = 32
    n_ch = rows_w // CH

    mesh = plsc.VectorSubcoreMesh(core_axis_name="c", subcore_axis_name="s")

    @functools.partial(
        pl.kernel,
        out_type=jax.ShapeDtypeStruct((B, S, D), jnp.float32),
        mesh=mesh,
        scratch_types=(
            [pltpu.VMEM((CH, D), jnp.float32)] * _NBUF
            + [pltpu.SemaphoreType.DMA] * (2 * _NBUF)
        ),
    )
    def sc_copy(emb_hbm, out_hbm, *scratch):
        bufs = scratch[:_NBUF]
        rsems = scratch[_NBUF:2 * _NBUF]
        wsems = scratch[2 * _NBUF:]
        wid = lax.axis_index("s") * _NC + lax.axis_index("c")
        base = wid * rows_w

        def read(i):
            r = base + i * CH
            return pltpu.make_async_copy(
                emb_hbm.at[pl.ds(r, CH)], bufs[i % _NBUF], rsems[i % _NBUF])

        def writes(i):
            r = base + i * CH
            return [
                pltpu.make_async_copy(
                    bufs[i % _NBUF], out_hbm.at[b, pl.ds(r, CH)],
                    wsems[i % _NBUF])
                for b in range(B)
            ]

        for k in range(min(_NBUF, n_ch)):
            read(k).start()
        drained = 0
        for i in range(n_ch):
            read(i).wait()
            for w in writes(i):
                w.start()
            j = i - _LAG
            if j >= 0 and j + _NBUF < n_ch:
                for w in writes(j):
                    w.wait()
                drained = j + 1
                read(j + _NBUF).start()
        for j in range(drained, n_ch):
            for w in writes(j):
                w.wait()

    return sc_copy(embedding[:S])
